# baseline (speedup 1.0000x reference)
import functools
import jax
import jax.numpy as jnp
from jax import lax
from jax.experimental import pallas as pl
from jax.experimental.pallas import tpu as pltpu
from jax.experimental.pallas import tpu_sc as plsc

D = 64
SCALE = 8.0
NC, NS = 2, 16
NW = NC * NS
RPB = 128
G = 8
C = G * RPB


@functools.lru_cache(maxsize=None)
def _build(B):
    rows_per_w = B // NW
    n_chunks = rows_per_w // C
    irows_per_w = rows_per_w // RPB
    mesh = plsc.VectorSubcoreMesh(core_axis_name="c", subcore_axis_name="s")

    @functools.partial(
        pl.kernel,
        out_type=jax.ShapeDtypeStruct((B, D), jnp.float32),
        mesh=mesh,
        compiler_params=pltpu.CompilerParams(use_tc_tiling_on_sc=False),
        scratch_types=[
            pltpu.VMEM((G, RPB), jnp.int32),
            pltpu.VMEM((C, D), jnp.float32),
            pltpu.SemaphoreType.DMA,
        ],
    )
    def emb(idx_hbm, table_hbm, out_hbm, idx_v, rows_v, sem):
        wid = lax.axis_index("s") * NC + lax.axis_index("c")
        irow0 = wid * irows_per_w
        base0 = wid * rows_per_w

        def chunk_body(g, carry):
            irow = irow0 + g * G
            base = base0 + g * C
            pltpu.sync_copy(idx_hbm.at[pl.ds(irow, G)], idx_v)
            handles = [
                pltpu.async_copy(
                    table_hbm.at[idx_v.at[j]],
                    rows_v.at[pl.ds(j * RPB, RPB)],
                    sem,
                )
                for j in range(G)
            ]
            for h in handles:
                h.wait()

            def scale_body(i, c2):
                for u in range(8):
                    r = i * 8 + u
                    for kk in range(4):
                        s = pl.ds(kk * 16, 16)
                        rows_v[r, s] = rows_v[r, s] * SCALE
                return c2

            lax.fori_loop(0, C // 8, scale_body, 0)
            pltpu.sync_copy(rows_v, out_hbm.at[pl.ds(base, C)])
            return carry

        lax.fori_loop(0, n_chunks, chunk_body, 0)

    return emb


def kernel(x, lut):
    B = x.size
    xi = x.reshape(B // RPB, RPB).astype(jnp.int32)
    out = _build(B)(xi, lut)
    return out.reshape(*x.shape, D)


# speedup vs baseline: 1.0079x; 1.0079x over previous
import functools
import jax
import jax.numpy as jnp
from jax import lax
from jax.experimental import pallas as pl
from jax.experimental.pallas import tpu as pltpu
from jax.experimental.pallas import tpu_sc as plsc

D = 64
SCALE = 8.0
NC, NS = 2, 16
NW = NC * NS
GR = 8
SPLITS = ((0, 104), (104, 96))


@functools.lru_cache(maxsize=None)
def _build(R, S):
    rows_per_w = R // NW
    n_chunks = rows_per_w // GR
    mesh = plsc.VectorSubcoreMesh(core_axis_name="c", subcore_axis_name="s")

    @functools.partial(
        pl.kernel,
        out_type=jax.ShapeDtypeStruct((R, S, D), jnp.float32),
        mesh=mesh,
        compiler_params=pltpu.CompilerParams(use_tc_tiling_on_sc=False),
        scratch_types=[
            pltpu.VMEM((GR, S), jnp.int32),
            pltpu.VMEM((GR, S, D), jnp.float32),
            pltpu.SemaphoreType.DMA,
        ],
    )
    def emb(idx_hbm, table_hbm, out_hbm, idx_v, rows_v, sem):
        wid = lax.axis_index("s") * NC + lax.axis_index("c")
        row0 = wid * rows_per_w

        def chunk_body(g, carry):
            r = row0 + g * GR
            pltpu.sync_copy(idx_hbm.at[pl.ds(r, GR)], idx_v)
            handles = []
            for j in range(GR):
                for off, ln in SPLITS:
                    handles.append(pltpu.async_copy(
                        table_hbm.at[idx_v.at[j, pl.ds(off, ln)]],
                        rows_v.at[j, pl.ds(off, ln)],
                        sem,
                    ))
            for hd in handles:
                hd.wait()

            def scale_body(c, c2):
                for j in range(GR):
                    for k in range(D // 16):
                        s = pl.ds(k * 16, 16)
                        rows_v[j, c, s] = rows_v[j, c, s] * SCALE
                return c2

            lax.fori_loop(0, S, scale_body, 0)
            pltpu.sync_copy(rows_v, out_hbm.at[pl.ds(r, GR)])
            return carry

        lax.fori_loop(0, n_chunks, chunk_body, 0)

    return emb


def kernel(x, lut):
    R, S = x.shape
    return _build(R, S)(x.astype(jnp.int32), lut)
